# baseline (speedup 1.0000x reference)
import functools

import jax
import jax.numpy as jnp
from jax import lax
from jax.experimental import pallas as pl
from jax.experimental.pallas import tpu as pltpu
from jax.experimental.pallas import tpu_sc as plsc

_B = 16384
_K = 64
_NC = 2
_NS = 16
_NW = _NC * _NS
_BPW = _B // _NW
_IDX_ROWS = _BPW // 128


def _sc_body(user_hbm, item_hbm, uidx_hbm, iidx_hbm, out_hbm,
             uidx_v, iidx_v, u_rows, i_rows, part, out_v, sem):
    wid = lax.axis_index("s") * _NC + lax.axis_index("c")
    base = wid * _BPW

    pltpu.sync_copy(uidx_hbm.at[pl.ds(wid * _IDX_ROWS, _IDX_ROWS)], uidx_v)
    pltpu.sync_copy(iidx_hbm.at[pl.ds(wid * _IDX_ROWS, _IDX_ROWS)], iidx_v)

    copies = []
    for j in range(_IDX_ROWS):
        copies.append(pltpu.async_copy(
            user_hbm.at[uidx_v.at[j]],
            u_rows.at[pl.ds(j * 128, 128)], sem))
        copies.append(pltpu.async_copy(
            item_hbm.at[iidx_v.at[j]],
            i_rows.at[pl.ds(j * 128, 128)], sem))
    for c in copies:
        c.wait()

    lane = lax.iota(jnp.int32, 16)

    def blk_body(blk, carry):
        rbase = blk * 16
        for ii in range(16):
            r = rbase + ii
            acc = u_rows[r, pl.ds(0, 16)] * i_rows[r, pl.ds(0, 16)]
            for k in range(1, _K // 16):
                acc = acc + (u_rows[r, pl.ds(16 * k, 16)] *
                             i_rows[r, pl.ds(16 * k, 16)])
            plsc.store_scatter(part, [lane * 16 + ii], acc)
        tot = part[pl.ds(0, 16)]
        for j in range(1, 16):
            tot = tot + part[pl.ds(j * 16, 16)]
        out_v[pl.ds(rbase, 16)] = 1.0 / (1.0 + jnp.exp(-tot))
        return carry

    lax.fori_loop(0, _BPW // 16, blk_body, 0)
    pltpu.sync_copy(out_v, out_hbm.at[pl.ds(base, _BPW)])


@functools.partial(jax.jit, static_argnums=())
def _mf_sc(user_emb_table, item_emb_table, uidx, iidx):
    mesh = plsc.VectorSubcoreMesh(core_axis_name="c", subcore_axis_name="s")
    run = pl.kernel(
        _sc_body,
        out_type=jax.ShapeDtypeStruct((_B,), jnp.float32),
        mesh=mesh,
        compiler_params=pltpu.CompilerParams(
            needs_layout_passes=False, use_tc_tiling_on_sc=False),
        scratch_types=[
            pltpu.VMEM((_IDX_ROWS, 128), jnp.int32),
            pltpu.VMEM((_IDX_ROWS, 128), jnp.int32),
            pltpu.VMEM((_BPW, _K), jnp.float32),
            pltpu.VMEM((_BPW, _K), jnp.float32),
            pltpu.VMEM((256,), jnp.float32),
            pltpu.VMEM((_BPW,), jnp.float32),
            pltpu.SemaphoreType.DMA,
        ],
    )
    return run(user_emb_table, item_emb_table, uidx, iidx)


def kernel(x, user_emb_table, item_emb_table):
    uidx = x[:, 0].astype(jnp.int32).reshape(_NW * _IDX_ROWS, 128)
    iidx = x[:, 1].astype(jnp.int32).reshape(_NW * _IDX_ROWS, 128)
    return _mf_sc(user_emb_table, item_emb_table, uidx, iidx)
